# baseline (speedup 1.0000x reference)
import functools

import jax
import jax.numpy as jnp
from jax import lax
from jax.experimental import pallas as pl
from jax.experimental.pallas import tpu as pltpu
from jax.experimental.pallas import tpu_sc as plsc

D = 64
B = 4096
L = 50
NW = 32
BAGS_PER_WORKER = B // NW
BAGS_PER_CHUNK = 16
CHUNKS = BAGS_PER_WORKER // BAGS_PER_CHUNK
ROWS_PER_CHUNK = BAGS_PER_CHUNK * L
GATHER_SPLIT = 8
ROWS_PER_GATHER = ROWS_PER_CHUNK // GATHER_SPLIT


def _sc_body(x_hbm, w_hbm, tbl_hbm, out_hbm, idx_v, rows_v, w_v, out_v, sem):
    wid = lax.axis_index("s") * 2 + lax.axis_index("c")
    iota = lax.iota(jnp.int32, 16)

    def chunk_body(it, carry):
        bag_base = wid * BAGS_PER_WORKER + it * BAGS_PER_CHUNK
        xrow = wid * (BAGS_PER_WORKER * L // ROWS_PER_GATHER) + it * GATHER_SPLIT
        pltpu.sync_copy(x_hbm.at[pl.ds(xrow, GATHER_SPLIT)], idx_v)
        copies = [
            pltpu.async_copy(
                tbl_hbm.at[idx_v.at[j]],
                rows_v.at[pl.ds(j * ROWS_PER_GATHER, ROWS_PER_GATHER)],
                sem,
            )
            for j in range(GATHER_SPLIT)
        ]
        pltpu.sync_copy(w_hbm.at[pl.ds(bag_base * L, ROWS_PER_CHUNK)], w_v)
        for c in copies:
            c.wait()

        for p in range(D // 16):
            def l_body(l, accs):
                base = iota * L + l
                wval = plsc.load_gather(w_v, [base])
                out = []
                for j in range(16):
                    dvec = jnp.full((16,), p * 16 + j, jnp.int32)
                    val = plsc.load_gather(rows_v, [base, dvec])
                    out.append(accs[j] + wval * val)
                return tuple(out)

            accs = lax.fori_loop(
                0, L, l_body,
                tuple(jnp.zeros((16,), jnp.float32) for _ in range(16)),
            )
            for j in range(16):
                dvec = jnp.full((16,), p * 16 + j, jnp.int32)
                plsc.store_scatter(out_v, [iota, dvec], accs[j])

        pltpu.sync_copy(out_v, out_hbm.at[pl.ds(bag_base, BAGS_PER_CHUNK)])
        return carry

    lax.fori_loop(0, CHUNKS, chunk_body, 0)


_sc_call = functools.partial(
    pl.kernel,
    out_type=jax.ShapeDtypeStruct((B, D), jnp.float32),
    mesh=plsc.VectorSubcoreMesh(core_axis_name="c", subcore_axis_name="s"),
    compiler_params=pltpu.CompilerParams(
        needs_layout_passes=False, use_tc_tiling_on_sc=False
    ),
    scratch_types=[
        pltpu.VMEM((GATHER_SPLIT, ROWS_PER_GATHER), jnp.int32),
        pltpu.VMEM((ROWS_PER_CHUNK, D), jnp.float32),
        pltpu.VMEM((ROWS_PER_CHUNK,), jnp.float32),
        pltpu.VMEM((BAGS_PER_CHUNK, D), jnp.float32),
        pltpu.SemaphoreType.DMA,
    ],
)(_sc_body)


def kernel(x, w, weight):
    x2 = x.astype(jnp.int32).reshape(B * L // ROWS_PER_GATHER, ROWS_PER_GATHER)
    return _sc_call(x2, w.reshape(B * L), weight)
